# Initial kernel scaffold; baseline (speedup 1.0000x reference)
import jax
import jax.numpy as jnp
from jax.experimental import pallas as pl


def kernel(x, split_idxs, split_vals, lookup_tables, bias):
    raise NotImplementedError("write your pallas kernel here")


# speedup vs baseline: 76.4906x; 76.4906x over previous
import functools

import jax
import jax.numpy as jnp
from jax.experimental import pallas as pl
from jax.experimental.pallas import tpu as pltpu

_NC = 32
_K = 16
_SUB = 64
_LEVELS = 4


def _body(x_ref, cols_ref, svp_ref, lut_ref, bias_ref, out_ref):
    rows = x_ref.shape[0]
    d = x_ref.shape[1]
    nc = _NC

    x = x_ref[...]
    iota_c = jax.lax.broadcasted_iota(jnp.int32, (d, nc * _LEVELS), 0)
    sel = (iota_c == cols_ref[...]).astype(jnp.float32)
    v = jax.lax.dot_general(
        x, sel, (((1,), (0,)), ((), ())),
        precision=jax.lax.Precision.HIGHEST,
        preferred_element_type=jnp.float32)

    g = jnp.zeros((rows, nc), jnp.int32)
    for level in range(_LEVELS):
        v_l = v[:, nc * level:nc * (level + 1)]
        thresh = jnp.zeros((rows, nc), jnp.float32)
        for b in range(1 << level):
            row = svp_ref[8 * level + b:8 * level + b + 1, :]
            thresh = thresh + jnp.where(g == b, row, 0.0)
        g = g * 2 + (v_l > thresh).astype(jnp.int32)

    iota_k = jax.lax.broadcasted_iota(jnp.int32, (rows, _K), 1)
    oh = jnp.concatenate(
        [(g[:, i:i + 1] == iota_k).astype(jnp.bfloat16) for i in range(nc)],
        axis=1)

    acc = jax.lax.dot_general(
        oh, lut_ref[...], (((1,), (0,)), ((), ())),
        preferred_element_type=jnp.float32)
    out_ref[...] = acc + bias_ref[...]


@functools.partial(jax.jit, static_argnames=())
def kernel(x, split_idxs, split_vals, lookup_tables, bias):
    n, d = x.shape
    nc, k, out_f = lookup_tables.shape
    sub = d // nc

    col_ids = (split_idxs.T.astype(jnp.int32)
               + sub * jnp.arange(nc, dtype=jnp.int32)[None, :]).reshape(1, _LEVELS * nc)
    svp = split_vals.transpose(1, 2, 0).reshape(_LEVELS * (k // 2), nc)
    lut_flat = lookup_tables.reshape(nc * k, out_f).astype(jnp.bfloat16)
    bias2 = bias.reshape(1, out_f)

    rows_blk = 512
    nb = n // rows_blk

    return pl.pallas_call(
        _body,
        grid=(nb,),
        in_specs=[
            pl.BlockSpec((rows_blk, d), lambda i: (i, 0)),
            pl.BlockSpec((1, _LEVELS * nc), lambda i: (0, 0)),
            pl.BlockSpec((_LEVELS * (k // 2), nc), lambda i: (0, 0)),
            pl.BlockSpec((nc * k, out_f), lambda i: (0, 0)),
            pl.BlockSpec((1, out_f), lambda i: (0, 0)),
        ],
        out_specs=pl.BlockSpec((rows_blk, out_f), lambda i: (i, 0)),
        out_shape=jax.ShapeDtypeStruct((n, out_f), jnp.float32),
        compiler_params=pltpu.CompilerParams(
            dimension_semantics=("arbitrary",)),
    )(x, col_ids, svp, lut_flat, bias2)


# speedup vs baseline: 105.1936x; 1.3752x over previous
import functools

import jax
import jax.numpy as jnp
from jax.experimental import pallas as pl
from jax.experimental.pallas import tpu as pltpu

_NC = 32
_K = 16
_SUB = 64
_LEVELS = 4


def _body(x_ref, sel_ref, svp_ref, lut_ref, bias_ref, out_ref):
    rows = x_ref.shape[0]
    nc = _NC

    x = x_ref[...]
    v = jax.lax.dot_general(
        x, sel_ref[...], (((1,), (0,)), ((), ())),
        precision=jax.lax.Precision.HIGHEST,
        preferred_element_type=jnp.float32)

    g = jnp.zeros((rows, nc), jnp.int32)
    for level in range(_LEVELS):
        v_l = v[:, nc * level:nc * (level + 1)]
        thresh = jnp.zeros((rows, nc), jnp.float32)
        for b in range(1 << level):
            row = svp_ref[8 * level + b:8 * level + b + 1, :]
            thresh = thresh + jnp.where(g == b, row, 0.0)
        g = g * 2 + (v_l > thresh).astype(jnp.int32)

    c = nc * _K
    exp_i = jax.lax.broadcasted_iota(jnp.int32, (nc, c), 0)
    exp_l = jax.lax.broadcasted_iota(jnp.int32, (nc, c), 1)
    expand = (exp_i == (exp_l // _K)).astype(jnp.bfloat16)
    e = jax.lax.dot_general(
        g.astype(jnp.bfloat16), expand, (((1,), (0,)), ((), ())),
        preferred_element_type=jnp.float32)
    lane_code = (jax.lax.broadcasted_iota(jnp.int32, (rows, c), 1)
                 % _K).astype(jnp.float32)
    oh = (e == lane_code).astype(jnp.bfloat16)

    acc = jax.lax.dot_general(
        oh, lut_ref[...], (((1,), (0,)), ((), ())),
        preferred_element_type=jnp.float32)
    out_ref[...] = acc + bias_ref[...]


@functools.partial(jax.jit, static_argnames=())
def kernel(x, split_idxs, split_vals, lookup_tables, bias):
    n, d = x.shape
    nc, k, out_f = lookup_tables.shape
    sub = d // nc

    col_ids = (split_idxs.T.astype(jnp.int32)
               + sub * jnp.arange(nc, dtype=jnp.int32)[None, :]).reshape(1, _LEVELS * nc)
    sel_mat = (jnp.arange(d, dtype=jnp.int32)[:, None]
               == col_ids).astype(jnp.float32)
    svp = split_vals.transpose(1, 2, 0).reshape(_LEVELS * (k // 2), nc)
    lut_flat = lookup_tables.reshape(nc * k, out_f).astype(jnp.bfloat16)
    bias2 = bias.reshape(1, out_f)

    rows_blk = 512
    nb = n // rows_blk

    return pl.pallas_call(
        _body,
        grid=(nb,),
        in_specs=[
            pl.BlockSpec((rows_blk, d), lambda i: (i, 0)),
            pl.BlockSpec((d, _LEVELS * nc), lambda i: (0, 0)),
            pl.BlockSpec((_LEVELS * (k // 2), nc), lambda i: (0, 0)),
            pl.BlockSpec((nc * k, out_f), lambda i: (0, 0)),
            pl.BlockSpec((1, out_f), lambda i: (0, 0)),
        ],
        out_specs=pl.BlockSpec((rows_blk, out_f), lambda i: (i, 0)),
        out_shape=jax.ShapeDtypeStruct((n, out_f), jnp.float32),
        compiler_params=pltpu.CompilerParams(
            dimension_semantics=("arbitrary",)),
    )(x, sel_mat, svp, lut_flat, bias2)


# speedup vs baseline: 105.2651x; 1.0007x over previous
import functools

import jax
import jax.numpy as jnp
from jax.experimental import pallas as pl
from jax.experimental.pallas import tpu as pltpu

_NC = 32
_K = 16
_SUB = 64
_LEVELS = 4


def _body(x_ref, sel_ref, svp_ref, lut_ref, bias_ref, out_ref):
    rows = x_ref.shape[0]
    nc = _NC

    x = x_ref[...]
    v = jax.lax.dot_general(
        x, sel_ref[...], (((1,), (0,)), ((), ())),
        precision=jax.lax.Precision.HIGHEST,
        preferred_element_type=jnp.float32)

    g = jnp.zeros((rows, nc), jnp.int32)
    for level in range(_LEVELS):
        v_l = v[:, nc * level:nc * (level + 1)]
        thresh = jnp.zeros((rows, nc), jnp.float32)
        for b in range(1 << level):
            row = svp_ref[8 * level + b:8 * level + b + 1, :]
            thresh = thresh + jnp.where(g == b, row, 0.0)
        g = g * 2 + (v_l > thresh).astype(jnp.int32)

    c = nc * _K
    exp_i = jax.lax.broadcasted_iota(jnp.int32, (nc, c), 0)
    exp_l = jax.lax.broadcasted_iota(jnp.int32, (nc, c), 1)
    expand = (exp_i == (exp_l // _K)).astype(jnp.bfloat16)
    e = jax.lax.dot_general(
        g.astype(jnp.bfloat16), expand, (((1,), (0,)), ((), ())),
        preferred_element_type=jnp.float32)
    lane_code = (jax.lax.broadcasted_iota(jnp.int32, (rows, c), 1)
                 % _K).astype(jnp.float32)
    oh = (e == lane_code).astype(jnp.bfloat16)

    acc = jax.lax.dot_general(
        oh, lut_ref[...], (((1,), (0,)), ((), ())),
        preferred_element_type=jnp.float32)
    out_ref[...] = acc + bias_ref[...]


@functools.partial(jax.jit, static_argnames=())
def kernel(x, split_idxs, split_vals, lookup_tables, bias):
    n, d = x.shape
    nc, k, out_f = lookup_tables.shape
    sub = d // nc

    col_ids = (split_idxs.T.astype(jnp.int32)
               + sub * jnp.arange(nc, dtype=jnp.int32)[None, :]).reshape(1, _LEVELS * nc)
    sel_mat = (jnp.arange(d, dtype=jnp.int32)[:, None]
               == col_ids).astype(jnp.float32)
    svp = split_vals.transpose(1, 2, 0).reshape(_LEVELS * (k // 2), nc)
    lut_flat = lookup_tables.reshape(nc * k, out_f).astype(jnp.bfloat16)
    bias2 = bias.reshape(1, out_f)

    rows_blk = 512
    nb = n // rows_blk

    return pl.pallas_call(
        _body,
        grid=(nb,),
        in_specs=[
            pl.BlockSpec((rows_blk, d), lambda i: (i, 0)),
            pl.BlockSpec((d, _LEVELS * nc), lambda i: (0, 0)),
            pl.BlockSpec((_LEVELS * (k // 2), nc), lambda i: (0, 0)),
            pl.BlockSpec((nc * k, out_f), lambda i: (0, 0)),
            pl.BlockSpec((1, out_f), lambda i: (0, 0)),
        ],
        out_specs=pl.BlockSpec((rows_blk, out_f), lambda i: (i, 0)),
        out_shape=jax.ShapeDtypeStruct((n, out_f), jnp.float32),
        compiler_params=pltpu.CompilerParams(
            dimension_semantics=("parallel",)),
    )(x, sel_mat, svp, lut_flat, bias2)
